# baseline (speedup 1.0000x reference)
import functools

import jax
import jax.numpy as jnp
from jax import lax
from jax.experimental import pallas as pl
from jax.experimental.pallas import tpu as pltpu
from jax.experimental.pallas import tpu_sc as plsc

_B, _N, _K, _T, _D = 64, 900, 100, 120, 256
_NC, _NS = 2, 16
_NW = _NC * _NS
_BPW = _B // _NW
_LANES = 16
_NCHUNK = (_K + _LANES - 1) // _LANES
_KP = _NCHUNK * _LANES
_KW = 104
_BIG = 1 << 20
_GB = 8

_mesh = plsc.VectorSubcoreMesh(core_axis_name="c", subcore_axis_name="s",
                               num_cores=_NC, num_subcores=_NS)


@functools.partial(
    pl.kernel,
    out_type=jax.ShapeDtypeStruct((_B, _KW, _D), jnp.float32),
    mesh=_mesh,
    compiler_params=pltpu.CompilerParams(needs_layout_passes=False,
                                         use_tc_tiling_on_sc=True),
    scratch_types=[
        pltpu.VMEM((_BPW, _K), jnp.int32),
        pltpu.VMEM((_BPW, _KP), jnp.int32),
        pltpu.VMEM((_BPW, _KP, _D), jnp.float32),
        pltpu.SemaphoreType.DMA,
    ],
)
def _sc_gather(hs_hbm, poi_hbm, hs_out, poi_v, gidx_v, hsrow_v, sem):
    w = lax.axis_index("s") * _NC + lax.axis_index("c")
    b0 = w * _BPW

    pltpu.sync_copy(poi_hbm.at[pl.ds(b0, _BPW)], poi_v)

    lane = lax.iota(jnp.int32, _LANES)
    for j in range(_BPW):
        jvec = jnp.full((_LANES,), j, jnp.int32)
        for c in range(_NCHUNK):
            pos_raw = lane + (_LANES * c)
            pos = jnp.minimum(pos_raw, _K - 1)
            ind = plsc.load_gather(poi_v, [jvec, pos])
            plsc.store_scatter(gidx_v, [jvec, pos_raw], ind)

    copies = [
        pltpu.async_copy(hs_hbm.at[b0 + j].at[gidx_v.at[j]],
                         hsrow_v.at[j], sem)
        for j in range(_BPW)
    ]
    for cp in copies:
        cp.wait()

    for j in range(_BPW):
        pltpu.sync_copy(hsrow_v.at[pl.ds(j, 1), pl.ds(0, _KW)],
                        hs_out.at[pl.ds(b0 + j, 1)])


def _tc_body(poi_ref, ptid_ref, tid_ref, box_ref,
             boxout_ref, ids_ref, unm_ref):
    ind = poi_ref[...]
    oh = (ind[:, :, None]
          == lax.broadcasted_iota(jnp.int32, (_GB, _K, _N), 2))
    boxout_ref[...] = jax.lax.dot_general(
        oh.astype(jnp.float32), box_ref[...],
        dimension_numbers=(((2,), (1,)), ((0,), (0,))),
        preferred_element_type=jnp.float32)

    prev = ptid_ref[...]
    trk = tid_ref[...]
    m = prev[:, :, None] == trk[:, None, :]
    tio = lax.broadcasted_iota(jnp.int32, (_GB, _K, _T), 2)
    first = jnp.min(jnp.where(m, tio, _BIG), axis=2)
    matched = first < _BIG
    ids_ref[...] = jnp.where(matched, first, 0)
    unm_ref[...] = jnp.where(matched, 0, 1)


_tc_match_boxes = pl.pallas_call(
    _tc_body,
    grid=(_B // _GB,),
    in_specs=[
        pl.BlockSpec((_GB, _K), lambda g: (g, 0)),
        pl.BlockSpec((_GB, _K), lambda g: (g, 0)),
        pl.BlockSpec((_GB, _T), lambda g: (g, 0)),
        pl.BlockSpec((_GB, _N, 4), lambda g: (g, 0, 0)),
    ],
    out_specs=[
        pl.BlockSpec((_GB, _K, 4), lambda g: (g, 0, 0)),
        pl.BlockSpec((_GB, _K), lambda g: (g, 0)),
        pl.BlockSpec((_GB, _K), lambda g: (g, 0)),
    ],
    out_shape=[
        jax.ShapeDtypeStruct((_B, _K, 4), jnp.float32),
        jax.ShapeDtypeStruct((_B, _K), jnp.int32),
        jax.ShapeDtypeStruct((_B, _K), jnp.int32),
    ],
)


def kernel(pred_boxes, hs_embed, prev_out_ind, prev_track_ids, track_ids):
    poi = prev_out_ind.astype(jnp.int32)
    ptid = prev_track_ids.astype(jnp.int32)
    tid = track_ids.astype(jnp.int32)

    hs_rows_p = _sc_gather(hs_embed, poi)
    box_rows, match_ids, unmatched = _tc_match_boxes(poi, ptid, tid, pred_boxes)
    hs_rows = hs_rows_p[:, :_K, :]

    zeros_n = jnp.zeros((_B, _N), dtype=bool)
    track_queries_mask = jnp.concatenate(
        [jnp.ones((_B, _K), dtype=bool), zeros_n], axis=1)
    track_queries_fal_pos_mask = jnp.concatenate(
        [unmatched.astype(bool), zeros_n], axis=1)
    return (
        hs_rows,
        box_rows,
        match_ids,
        track_queries_mask,
        track_queries_fal_pos_mask,
    )


# speedup vs baseline: 1.0051x; 1.0051x over previous
import functools

import jax
import jax.numpy as jnp
from jax import lax
from jax.experimental import pallas as pl
from jax.experimental.pallas import tpu as pltpu
from jax.experimental.pallas import tpu_sc as plsc

_B, _N, _K, _T, _D = 64, 900, 100, 120, 256
_NC, _NS = 2, 16
_NW = _NC * _NS
_BPW = _B // _NW
_LANES = 16
_NCHUNK = (_K + _LANES - 1) // _LANES
_KP = _NCHUNK * _LANES
_KW = 104
_BIG = 1 << 20
_GB = 8

_mesh = plsc.VectorSubcoreMesh(core_axis_name="c", subcore_axis_name="s",
                               num_cores=_NC, num_subcores=_NS)


@functools.partial(
    pl.kernel,
    out_type=jax.ShapeDtypeStruct((_B, _KW, _D), jnp.float32),
    mesh=_mesh,
    compiler_params=pltpu.CompilerParams(needs_layout_passes=False,
                                         use_tc_tiling_on_sc=True),
    scratch_types=[
        pltpu.VMEM((_BPW, _K), jnp.int32),
        pltpu.VMEM((_BPW, _KP), jnp.int32),
        pltpu.VMEM((_BPW, _KP, _D), jnp.float32),
        pltpu.SemaphoreType.DMA,
    ],
)
def _sc_gather(hs_hbm, poi_hbm, hs_out, poi_v, gidx_v, hsrow_v, sem):
    w = lax.axis_index("s") * _NC + lax.axis_index("c")
    b0 = w * _BPW

    pltpu.sync_copy(poi_hbm.at[pl.ds(b0, _BPW)], poi_v)

    lane = lax.iota(jnp.int32, _LANES)
    for j in range(_BPW):
        jvec = jnp.full((_LANES,), j, jnp.int32)
        for c in range(_NCHUNK):
            pos_raw = lane + (_LANES * c)
            pos = jnp.minimum(pos_raw, _K - 1)
            ind = plsc.load_gather(poi_v, [jvec, pos])
            plsc.store_scatter(gidx_v, [jvec, pos_raw], ind)

    copies = [
        pltpu.async_copy(hs_hbm.at[b0 + j].at[gidx_v.at[j]],
                         hsrow_v.at[j], sem)
        for j in range(_BPW)
    ]
    for cp in copies:
        cp.wait()

    for j in range(_BPW):
        pltpu.sync_copy(hsrow_v.at[pl.ds(j, 1), pl.ds(0, _KW)],
                        hs_out.at[pl.ds(b0 + j, 1)])


def _tc_body(poi_ref, ptid_ref, tid_ref, box_ref,
             boxout_ref, ids_ref, unm_ref):
    ind = poi_ref[...]
    oh = (ind[:, :, None]
          == lax.broadcasted_iota(jnp.int32, (_GB, _K, _N), 2))
    boxout_ref[...] = jax.lax.dot_general(
        oh.astype(jnp.float32), box_ref[...],
        dimension_numbers=(((2,), (1,)), ((0,), (0,))),
        preferred_element_type=jnp.float32)

    prev = ptid_ref[...]
    trk = tid_ref[...]
    m = prev[:, :, None] == trk[:, None, :]
    tio = lax.broadcasted_iota(jnp.int32, (_GB, _K, _T), 2)
    first = jnp.min(jnp.where(m, tio, _BIG), axis=2)
    matched = first < _BIG
    ids_ref[...] = jnp.where(matched, first, 0)
    unm_ref[...] = jnp.where(matched, 0, 1)


_tc_match_boxes = pl.pallas_call(
    _tc_body,
    grid=(_B // _GB,),
    in_specs=[
        pl.BlockSpec((_GB, _K), lambda g: (g, 0)),
        pl.BlockSpec((_GB, _K), lambda g: (g, 0)),
        pl.BlockSpec((_GB, _T), lambda g: (g, 0)),
        pl.BlockSpec((_GB, _N, 4), lambda g: (g, 0, 0)),
    ],
    out_specs=[
        pl.BlockSpec((_GB, _K, 4), lambda g: (g, 0, 0)),
        pl.BlockSpec((_GB, _K), lambda g: (g, 0)),
        pl.BlockSpec((_GB, _K), lambda g: (g, 0)),
    ],
    out_shape=[
        jax.ShapeDtypeStruct((_B, _K, 4), jnp.float32),
        jax.ShapeDtypeStruct((_B, _K), jnp.int32),
        jax.ShapeDtypeStruct((_B, _K), jnp.int32),
    ],
)


def kernel(pred_boxes, hs_embed, prev_out_ind, prev_track_ids, track_ids):
    poi = prev_out_ind.astype(jnp.int32)
    ptid = prev_track_ids.astype(jnp.int32)
    tid = track_ids.astype(jnp.int32)

    box_rows, match_ids, unmatched = _tc_match_boxes(poi, ptid, tid, pred_boxes)
    hs_rows_p = _sc_gather(hs_embed, poi)
    hs_rows = hs_rows_p[:, :_K, :]

    zeros_n = jnp.zeros((_B, _N), dtype=bool)
    track_queries_mask = jnp.concatenate(
        [jnp.ones((_B, _K), dtype=bool), zeros_n], axis=1)
    track_queries_fal_pos_mask = jnp.concatenate(
        [unmatched.astype(bool), zeros_n], axis=1)
    return (
        hs_rows,
        box_rows,
        match_ids,
        track_queries_mask,
        track_queries_fal_pos_mask,
    )


# speedup vs baseline: 2.3957x; 2.3835x over previous
import functools

import jax
import jax.numpy as jnp
from jax import lax
from jax.experimental import pallas as pl
from jax.experimental.pallas import tpu as pltpu
from jax.experimental.pallas import tpu_sc as plsc

_B, _N, _K, _T, _D = 64, 900, 100, 120, 256
_NC, _NS = 2, 16
_NW = _NC * _NS
_BPW = _B // _NW
_LANES = 16
_NCHUNK = (_K + _LANES - 1) // _LANES
_KP = _NCHUNK * _LANES
_BIG = 1 << 20

_mesh = plsc.VectorSubcoreMesh(core_axis_name="c", subcore_axis_name="s",
                               num_cores=_NC, num_subcores=_NS)


@functools.partial(
    pl.kernel,
    out_type=(
        jax.ShapeDtypeStruct((_K * _B, _D), jnp.float32),
        jax.ShapeDtypeStruct((_B * 4 * _K,), jnp.float32),
        jax.ShapeDtypeStruct((_B, _K), jnp.int32),
        jax.ShapeDtypeStruct((_B, _K), jnp.int32),
    ),
    mesh=_mesh,
    compiler_params=pltpu.CompilerParams(needs_layout_passes=False),
    scratch_types=[
        pltpu.VMEM((_BPW, _K), jnp.int32),
        pltpu.VMEM((_BPW, _K), jnp.int32),
        pltpu.VMEM((_BPW, _T), jnp.int32),
        pltpu.VMEM((_BPW, _KP), jnp.int32),
        pltpu.VMEM((_BPW, _KP), jnp.int32),
        pltpu.VMEM((_BPW, _KP, _D), jnp.float32),
        pltpu.VMEM((_BPW * 4 * _N,), jnp.float32),
        pltpu.VMEM((_BPW * 4 * _K,), jnp.float32),
        pltpu.VMEM((_BPW, _K), jnp.int32),
        pltpu.VMEM((_BPW, _K), jnp.int32),
        pltpu.SemaphoreType.DMA,
        pltpu.SemaphoreType.DMA,
    ],
)
def _sc_all(hs_hbm, box_hbm, poi_hbm, ptid_hbm, tid_hbm,
            hs_out, box_out, ids_out, unm_out,
            poi_v, ptid_v, tid_v, gidx_v, oidx_v, hsrow_v,
            boxstage_v, boxrow_v, ids_v, unm_v, gsem, ssem):
    w = lax.axis_index("s") * _NC + lax.axis_index("c")
    b0 = w * _BPW

    pltpu.sync_copy(poi_hbm.at[pl.ds(b0, _BPW)], poi_v)
    pltpu.sync_copy(ptid_hbm.at[pl.ds(b0, _BPW)], ptid_v)
    pltpu.sync_copy(tid_hbm.at[pl.ds(b0, _BPW)], tid_v)
    pltpu.sync_copy(box_hbm.at[pl.ds(b0 * 4 * _N, _BPW * 4 * _N)], boxstage_v)

    lane = lax.iota(jnp.int32, _LANES)

    for j in range(_BPW):
        jvec = jnp.full((_LANES,), j, jnp.int32)
        b = b0 + j
        for c in range(_NCHUNK):
            pos_raw = lane + (_LANES * c)
            pos = jnp.minimum(pos_raw, _K - 1)
            ind = plsc.load_gather(poi_v, [jvec, pos])
            plsc.store_scatter(gidx_v, [jvec, pos_raw], ind * _B + b)
            plsc.store_scatter(oidx_v, [jvec, pos_raw], pos * _B + b)

    gathers = [
        pltpu.async_copy(hs_hbm.at[gidx_v.at[j]], hsrow_v.at[j], gsem)
        for j in range(_BPW)
    ]

    for j in range(_BPW):
        jvec = jnp.full((_LANES,), j, jnp.int32)
        for c in range(_NCHUNK):
            pos_raw = lane + (_LANES * c)
            valid = pos_raw < _K
            pos = jnp.minimum(pos_raw, _K - 1)
            ind = plsc.load_gather(poi_v, [jvec, pos])
            for comp in range(4):
                src = (j * 4 + comp) * _N + ind
                dst = (j * 4 + comp) * _K + pos
                val = plsc.load_gather(boxstage_v, [src])
                plsc.store_scatter(boxrow_v, [dst], val, mask=valid)

    for j in range(_BPW):
        jvec = jnp.full((_LANES,), j, jnp.int32)
        pvs = []
        for c in range(_NCHUNK):
            pos = jnp.minimum(lane + (_LANES * c), _K - 1)
            pvs.append(plsc.load_gather(ptid_v, [jvec, pos]))

        def t_chunk(base, accs, _j=j, _pvs=pvs):
            for i in range(_LANES):
                tvi = plsc.load_gather(
                    tid_v, [jnp.full((_LANES,), _j, jnp.int32),
                            base + jnp.full((_LANES,), i, jnp.int32)])
                tval = base + i
                accs = tuple(
                    jnp.minimum(a, jnp.where(p == tvi, tval, _BIG))
                    for a, p in zip(accs, _pvs)
                )
            return accs

        init = tuple(jnp.full((_LANES,), _BIG, jnp.int32) for _ in range(_NCHUNK))
        accs = lax.fori_loop(0, _T // _LANES,
                             lambda tc, accs: t_chunk(tc * _LANES, accs), init)
        if _T % _LANES:
            accs = t_chunk(_T - _LANES, accs)

        for c in range(_NCHUNK):
            acc = accs[c]
            matched = acc < _BIG
            pos_raw = lane + (_LANES * c)
            valid = pos_raw < _K
            pos = jnp.minimum(pos_raw, _K - 1)
            plsc.store_scatter(ids_v, [jvec, pos],
                               jnp.where(matched, acc, 0), mask=valid)
            plsc.store_scatter(unm_v, [jvec, pos],
                               jnp.where(matched, 0, 1), mask=valid)

    scatters = []
    for j in range(_BPW):
        gathers[j].wait()
        scatters.append(
            pltpu.async_copy(hsrow_v.at[j], hs_out.at[oidx_v.at[j]], ssem))

    pltpu.sync_copy(boxrow_v, box_out.at[pl.ds(b0 * 4 * _K, _BPW * 4 * _K)])
    pltpu.sync_copy(ids_v, ids_out.at[pl.ds(b0, _BPW)])
    pltpu.sync_copy(unm_v, unm_out.at[pl.ds(b0, _BPW)])
    for cp in scatters:
        cp.wait()


def kernel(pred_boxes, hs_embed, prev_out_ind, prev_track_ids, track_ids):
    hs_flat = hs_embed.transpose(1, 0, 2).reshape(_N * _B, _D)
    box_flat = pred_boxes.transpose(0, 2, 1).reshape(_B * 4 * _N)
    poi = prev_out_ind.astype(jnp.int32)
    ptid = prev_track_ids.astype(jnp.int32)
    tid = track_ids.astype(jnp.int32)

    hs_kb, box_bck, match_ids, unmatched = _sc_all(
        hs_flat, box_flat, poi, ptid, tid)

    hs_rows = hs_kb.reshape(_K, _B, _D).transpose(1, 0, 2)
    box_rows = box_bck.reshape(_B, 4, _K).transpose(0, 2, 1)

    zeros_n = jnp.zeros((_B, _N), dtype=bool)
    track_queries_mask = jnp.concatenate(
        [jnp.ones((_B, _K), dtype=bool), zeros_n], axis=1)
    track_queries_fal_pos_mask = jnp.concatenate(
        [unmatched.astype(bool), zeros_n], axis=1)
    return (
        hs_rows,
        box_rows,
        match_ids,
        track_queries_mask,
        track_queries_fal_pos_mask,
    )
